# baseline (speedup 1.0000x reference)
import functools
import math

import jax
import jax.numpy as jnp
from jax import lax
from jax.experimental import pallas as pl
from jax.experimental.pallas import tpu as pltpu
from jax.experimental.pallas import tpu_sc as plsc

S = 2048
D = 1024
E = 8
FF = 2048
T = 128
SPAD = S + E * T
NT = SPAD // T
RB = 256
NRB = S // RB

NC, NS = 2, 16
NW = NC * NS
CHUNK = S // NW


def _router_body(x_ref, rw_ref, rb_ref, eid_ref, rank_ref, cnt_ref, carry_ref):
    i = pl.program_id(0)

    @pl.when(i == 0)
    def _():
        carry_ref[...] = jnp.zeros_like(carry_ref)

    logits = jnp.dot(x_ref[...], rw_ref[...],
                     preferred_element_type=jnp.float32) + rb_ref[...]
    m = jnp.max(logits, axis=1, keepdims=True)
    ex = jnp.exp(logits - m)
    probs = ex / jnp.sum(ex, axis=1, keepdims=True)
    eid = jnp.argmax(probs, axis=1, keepdims=True).astype(jnp.int32)

    eiota = lax.broadcasted_iota(jnp.int32, (RB, E), 1)
    onehot = (eid == eiota).astype(jnp.float32)

    row = lax.broadcasted_iota(jnp.int32, (RB, RB), 0)
    col = lax.broadcasted_iota(jnp.int32, (RB, RB), 1)
    tri = (row > col).astype(jnp.float32)
    carry = carry_ref[0:1, 0:E]
    ranks = jnp.dot(tri, onehot, preferred_element_type=jnp.float32) + carry
    rank = jnp.sum(ranks * onehot, axis=1, keepdims=True)

    new_carry = carry + jnp.sum(onehot, axis=0, keepdims=True)
    carry_ref[0:1, 0:E] = new_carry

    eid_ref[0, :, :] = eid
    rank_ref[0, :, :] = rank.astype(jnp.int32)
    cnt_ref[0, :, :] = new_carry.astype(jnp.int32)


def _run_router(x2, route_W, route_b):
    return pl.pallas_call(
        _router_body,
        grid=(NRB,),
        in_specs=[
            pl.BlockSpec((RB, D), lambda i: (i, 0)),
            pl.BlockSpec((D, E), lambda i: (0, 0)),
            pl.BlockSpec((1, E), lambda i: (0, 0)),
        ],
        out_specs=[
            pl.BlockSpec((1, RB, 1), lambda i: (i, 0, 0)),
            pl.BlockSpec((1, RB, 1), lambda i: (i, 0, 0)),
            pl.BlockSpec((1, 1, E), lambda i: (0, 0, 0)),
        ],
        out_shape=[
            jax.ShapeDtypeStruct((NRB, RB, 1), jnp.int32),
            jax.ShapeDtypeStruct((NRB, RB, 1), jnp.int32),
            jax.ShapeDtypeStruct((1, 1, E), jnp.int32),
        ],
        scratch_shapes=[pltpu.VMEM((8, 128), jnp.float32)],
    )(x2, route_W, route_b.reshape(1, E))


def _dispatch_body(eid_ref, rank_ref, cnt_ref, p_ref, te_ref):
    cnt = cnt_ref[...]
    padded = ((cnt + (T - 1)) // T) * T
    pf = padded.astype(jnp.float32)
    r = lax.broadcasted_iota(jnp.int32, (E, E), 0)
    c = lax.broadcasted_iota(jnp.int32, (E, E), 1)
    le = (r <= c).astype(jnp.float32)
    incl = jnp.dot(pf, le, preferred_element_type=jnp.float32)
    excl_i = (incl - pf).astype(jnp.int32)
    incl_i = incl.astype(jnp.int32)

    eids = eid_ref[...]
    sel = lax.broadcasted_iota(jnp.int32, (1, E), 1)
    p = rank_ref[...]
    tstart = lax.broadcasted_iota(jnp.int32, (1, 128), 1) * T
    te = jnp.zeros((1, 128), jnp.int32)
    for e in range(E):
        off_e = jnp.sum(jnp.where(sel == e, excl_i, 0))
        p = p + jnp.where(eids == e, off_e, 0)
        end_e = jnp.sum(jnp.where(sel == e, incl_i, 0))
        te = te + (tstart >= end_e).astype(jnp.int32)
    p_ref[...] = p
    te_ref[...] = jnp.minimum(te, E - 1)


def _run_dispatch(eids2, ranks2, cnt2):
    return pl.pallas_call(
        _dispatch_body,
        in_specs=[
            pl.BlockSpec((16, 128), lambda: (0, 0)),
            pl.BlockSpec((16, 128), lambda: (0, 0)),
            pl.BlockSpec((1, E), lambda: (0, 0)),
        ],
        out_specs=[
            pl.BlockSpec((16, 128), lambda: (0, 0)),
            pl.BlockSpec((1, 128), lambda: (0, 0)),
        ],
        out_shape=[
            jax.ShapeDtypeStruct((16, 128), jnp.int32),
            jax.ShapeDtypeStruct((1, 128), jnp.int32),
        ],
    )(eids2, ranks2, cnt2)


def _make_scatter():
    mesh = plsc.VectorSubcoreMesh(core_axis_name="c", subcore_axis_name="s")

    @functools.partial(
        pl.kernel,
        mesh=mesh,
        out_type=jax.ShapeDtypeStruct((SPAD, D), jnp.float32),
        scratch_types=[
            pltpu.VMEM((CHUNK,), jnp.int32),
            pltpu.VMEM((CHUNK, D), jnp.float32),
            pltpu.SemaphoreType.DMA,
        ],
    )
    def scatter_k(x_hbm, p_hbm, xs_hbm, idx_v, rows_v, sem):
        wid = lax.axis_index("s") * NC + lax.axis_index("c")
        base = wid * CHUNK
        pltpu.sync_copy(p_hbm.at[pl.ds(base, CHUNK)], idx_v)
        pltpu.sync_copy(x_hbm.at[pl.ds(base, CHUNK)], rows_v)
        pltpu.async_copy(rows_v, xs_hbm.at[idx_v], sem).wait()

    return scatter_k


def _ffn_body(te_ref, xs_ref, w1_ref, b1_ref, w2_ref, b2_ref, out_ref):
    xb = xs_ref[...].astype(jnp.bfloat16)
    h = jnp.dot(xb, w1_ref[0],
                preferred_element_type=jnp.float32) + b1_ref[0]
    g = 0.5 * h * (1.0 + lax.erf(h * (1.0 / math.sqrt(2.0))))
    out_ref[...] = jnp.dot(g.astype(jnp.bfloat16), w2_ref[0],
                           preferred_element_type=jnp.float32) + b2_ref[0]


def _run_ffn(te, xs, W1, b1, W2, b2):
    grid_spec = pltpu.PrefetchScalarGridSpec(
        num_scalar_prefetch=1,
        grid=(NT,),
        in_specs=[
            pl.BlockSpec((T, D), lambda t, te: (t, 0)),
            pl.BlockSpec((1, D, FF), lambda t, te: (te[t], 0, 0)),
            pl.BlockSpec((1, 1, FF), lambda t, te: (te[t], 0, 0)),
            pl.BlockSpec((1, FF, D), lambda t, te: (te[t], 0, 0)),
            pl.BlockSpec((1, 1, D), lambda t, te: (te[t], 0, 0)),
        ],
        out_specs=pl.BlockSpec((T, D), lambda t, te: (t, 0)),
    )
    return pl.pallas_call(
        _ffn_body,
        grid_spec=grid_spec,
        out_shape=jax.ShapeDtypeStruct((SPAD, D), jnp.float32),
        compiler_params=pltpu.CompilerParams(
            vmem_limit_bytes=100 * 1024 * 1024),
    )(te, xs, W1.astype(jnp.bfloat16), b1.reshape(E, 1, FF),
      W2.astype(jnp.bfloat16), b2.reshape(E, 1, D))


def _make_gather():
    mesh = plsc.VectorSubcoreMesh(core_axis_name="c", subcore_axis_name="s")

    @functools.partial(
        pl.kernel,
        mesh=mesh,
        out_type=jax.ShapeDtypeStruct((S, D), jnp.float32),
        scratch_types=[
            pltpu.VMEM((CHUNK,), jnp.int32),
            pltpu.VMEM((CHUNK, D), jnp.float32),
            pltpu.SemaphoreType.DMA,
        ],
    )
    def gather_k(ys_hbm, p_hbm, out_hbm, idx_v, rows_v, sem):
        wid = lax.axis_index("s") * NC + lax.axis_index("c")
        base = wid * CHUNK
        pltpu.sync_copy(p_hbm.at[pl.ds(base, CHUNK)], idx_v)
        pltpu.async_copy(ys_hbm.at[idx_v], rows_v, sem).wait()
        pltpu.sync_copy(rows_v, out_hbm.at[pl.ds(base, CHUNK)])

    return gather_k


_scatter_k = _make_scatter()
_gather_k = _make_gather()


def kernel(x, route_W, route_b, W1, b1, W2, b2):
    x2 = x.reshape(S, D)
    eid3, rank3, cnt3 = _run_router(x2, route_W, route_b)
    eids2 = eid3.reshape(16, 128)
    ranks2 = rank3.reshape(16, 128)
    p2, te2 = _run_dispatch(eids2, ranks2, cnt3.reshape(1, E))
    p = p2.reshape(S)
    te = te2.reshape(128)
    xs = _scatter_k(x2, p)
    ys = _run_ffn(te, xs, W1, b1, W2, b2)
    out = _gather_k(ys, p)
    return out.reshape(1, S, D)


# speedup vs baseline: 1.3599x; 1.3599x over previous
import functools
import math

import jax
import jax.numpy as jnp
from jax import lax
from jax.experimental import pallas as pl
from jax.experimental.pallas import tpu as pltpu
from jax.experimental.pallas import tpu_sc as plsc

S = 2048
D = 1024
E = 8
FF = 2048
T = 128
SPAD = S + E * T
NT = SPAD // T
RB = 256
NRB = S // RB

NC, NS = 2, 16
NW = NC * NS
CHUNK = S // NW


def _router_body(x_ref, rw_ref, rb_ref, eid_ref, rank_ref, cnt_ref, carry_ref):
    i = pl.program_id(0)

    @pl.when(i == 0)
    def _():
        carry_ref[...] = jnp.zeros_like(carry_ref)

    logits = jnp.dot(x_ref[...], rw_ref[...],
                     preferred_element_type=jnp.float32) + rb_ref[...]
    m = jnp.max(logits, axis=1, keepdims=True)
    ex = jnp.exp(logits - m)
    probs = ex / jnp.sum(ex, axis=1, keepdims=True)
    eid = jnp.argmax(probs, axis=1, keepdims=True).astype(jnp.int32)

    eiota = lax.broadcasted_iota(jnp.int32, (RB, E), 1)
    onehot = (eid == eiota).astype(jnp.float32)

    row = lax.broadcasted_iota(jnp.int32, (RB, RB), 0)
    col = lax.broadcasted_iota(jnp.int32, (RB, RB), 1)
    tri = (row > col).astype(jnp.float32)
    carry = carry_ref[0:1, 0:E]
    ranks = jnp.dot(tri, onehot, preferred_element_type=jnp.float32) + carry
    rank = jnp.sum(ranks * onehot, axis=1, keepdims=True)

    new_carry = carry + jnp.sum(onehot, axis=0, keepdims=True)
    carry_ref[0:1, 0:E] = new_carry

    eid_ref[0, :, :] = eid
    rank_ref[0, :, :] = rank.astype(jnp.int32)
    cnt_ref[0, :, :] = new_carry.astype(jnp.int32)


def _run_router(x2, route_W, route_b):
    return pl.pallas_call(
        _router_body,
        grid=(NRB,),
        in_specs=[
            pl.BlockSpec((RB, D), lambda i: (i, 0)),
            pl.BlockSpec((D, E), lambda i: (0, 0)),
            pl.BlockSpec((1, E), lambda i: (0, 0)),
        ],
        out_specs=[
            pl.BlockSpec((1, RB, 1), lambda i: (i, 0, 0)),
            pl.BlockSpec((1, RB, 1), lambda i: (i, 0, 0)),
            pl.BlockSpec((1, 1, E), lambda i: (0, 0, 0)),
        ],
        out_shape=[
            jax.ShapeDtypeStruct((NRB, RB, 1), jnp.int32),
            jax.ShapeDtypeStruct((NRB, RB, 1), jnp.int32),
            jax.ShapeDtypeStruct((1, 1, E), jnp.int32),
        ],
        scratch_shapes=[pltpu.VMEM((8, 128), jnp.float32)],
    )(x2, route_W, route_b.reshape(1, E))


def _dispatch_body(eid_ref, rank_ref, cnt_ref, p_ref, te_ref, valid_ref):
    cnt = cnt_ref[...]
    padded = ((cnt + (T - 1)) // T) * T
    pf = padded.astype(jnp.float32)
    r = lax.broadcasted_iota(jnp.int32, (E, E), 0)
    c = lax.broadcasted_iota(jnp.int32, (E, E), 1)
    le = (r <= c).astype(jnp.float32)
    incl = jnp.dot(pf, le, preferred_element_type=jnp.float32)
    excl_i = (incl - pf).astype(jnp.int32)
    incl_i = incl.astype(jnp.int32)

    eids = eid_ref[...]
    sel = lax.broadcasted_iota(jnp.int32, (1, E), 1)
    p = rank_ref[...]
    tstart = lax.broadcasted_iota(jnp.int32, (1, 128), 1) * T
    te = jnp.zeros((1, 128), jnp.int32)
    for e in range(E):
        off_e = jnp.sum(jnp.where(sel == e, excl_i, 0))
        p = p + jnp.where(eids == e, off_e, 0)
        end_e = jnp.sum(jnp.where(sel == e, incl_i, 0))
        te = te + (tstart >= end_e).astype(jnp.int32)
    p_ref[...] = p
    te_ref[...] = jnp.minimum(te, E - 1)
    valid_ref[...] = (tstart < jnp.max(incl_i)).astype(jnp.int32)


def _run_dispatch(eids2, ranks2, cnt2):
    return pl.pallas_call(
        _dispatch_body,
        in_specs=[
            pl.BlockSpec((16, 128), lambda: (0, 0)),
            pl.BlockSpec((16, 128), lambda: (0, 0)),
            pl.BlockSpec((1, E), lambda: (0, 0)),
        ],
        out_specs=[
            pl.BlockSpec((16, 128), lambda: (0, 0)),
            pl.BlockSpec((1, 128), lambda: (0, 0)),
            pl.BlockSpec((1, 128), lambda: (0, 0)),
        ],
        out_shape=[
            jax.ShapeDtypeStruct((16, 128), jnp.int32),
            jax.ShapeDtypeStruct((1, 128), jnp.int32),
            jax.ShapeDtypeStruct((1, 128), jnp.int32),
        ],
    )(eids2, ranks2, cnt2)


def _make_scatter():
    mesh = plsc.VectorSubcoreMesh(core_axis_name="c", subcore_axis_name="s")

    @functools.partial(
        pl.kernel,
        mesh=mesh,
        out_type=jax.ShapeDtypeStruct((SPAD, D), jnp.float32),
        scratch_types=[
            pltpu.VMEM((CHUNK,), jnp.int32),
            pltpu.VMEM((CHUNK, D), jnp.float32),
            pltpu.SemaphoreType.DMA,
        ],
    )
    def scatter_k(x_hbm, p_hbm, xs_hbm, idx_v, rows_v, sem):
        wid = lax.axis_index("s") * NC + lax.axis_index("c")
        base = wid * CHUNK
        pltpu.sync_copy(p_hbm.at[pl.ds(base, CHUNK)], idx_v)
        pltpu.sync_copy(x_hbm.at[pl.ds(base, CHUNK)], rows_v)
        pltpu.async_copy(rows_v, xs_hbm.at[idx_v], sem).wait()

    return scatter_k


def _ffn_body(valid_ref, te_ref, xs_ref, w1_ref, b1_ref, w2_ref, b2_ref,
              out_ref):
    t = pl.program_id(0)

    @pl.when(valid_ref[t] != 0)
    def _():
        h = jnp.dot(xs_ref[...], w1_ref[0],
                    preferred_element_type=jnp.float32) + b1_ref[0]
        g = 0.5 * h * (1.0 + lax.erf(h * (1.0 / math.sqrt(2.0))))
        out_ref[...] = jnp.dot(g, w2_ref[0],
                               preferred_element_type=jnp.float32) + b2_ref[0]


def _run_ffn(valid, te, xs, W1, b1, W2, b2):
    grid_spec = pltpu.PrefetchScalarGridSpec(
        num_scalar_prefetch=2,
        grid=(NT,),
        in_specs=[
            pl.BlockSpec((T, D), lambda t, v, te: (t, 0)),
            pl.BlockSpec((1, D, FF), lambda t, v, te: (te[t], 0, 0)),
            pl.BlockSpec((1, 1, FF), lambda t, v, te: (te[t], 0, 0)),
            pl.BlockSpec((1, FF, D), lambda t, v, te: (te[t], 0, 0)),
            pl.BlockSpec((1, 1, D), lambda t, v, te: (te[t], 0, 0)),
        ],
        out_specs=pl.BlockSpec((T, D), lambda t, v, te: (t, 0)),
    )
    return pl.pallas_call(
        _ffn_body,
        grid_spec=grid_spec,
        out_shape=jax.ShapeDtypeStruct((SPAD, D), jnp.float32),
        compiler_params=pltpu.CompilerParams(
            vmem_limit_bytes=100 * 1024 * 1024),
    )(valid, te, xs, W1, b1.reshape(E, 1, FF), W2, b2.reshape(E, 1, D))


def _make_gather():
    mesh = plsc.VectorSubcoreMesh(core_axis_name="c", subcore_axis_name="s")

    @functools.partial(
        pl.kernel,
        mesh=mesh,
        out_type=jax.ShapeDtypeStruct((S, D), jnp.float32),
        scratch_types=[
            pltpu.VMEM((CHUNK,), jnp.int32),
            pltpu.VMEM((CHUNK, D), jnp.float32),
            pltpu.SemaphoreType.DMA,
        ],
    )
    def gather_k(ys_hbm, p_hbm, out_hbm, idx_v, rows_v, sem):
        wid = lax.axis_index("s") * NC + lax.axis_index("c")
        base = wid * CHUNK
        pltpu.sync_copy(p_hbm.at[pl.ds(base, CHUNK)], idx_v)
        pltpu.async_copy(ys_hbm.at[idx_v], rows_v, sem).wait()
        pltpu.sync_copy(rows_v, out_hbm.at[pl.ds(base, CHUNK)])

    return gather_k


_scatter_k = _make_scatter()
_gather_k = _make_gather()


def kernel(x, route_W, route_b, W1, b1, W2, b2):
    x2 = x.reshape(S, D)
    eid3, rank3, cnt3 = _run_router(x2, route_W, route_b)
    eids2 = eid3.reshape(16, 128)
    ranks2 = rank3.reshape(16, 128)
    p2, te2, valid2 = _run_dispatch(eids2, ranks2, cnt3.reshape(1, E))
    p = p2.reshape(S)
    te = te2.reshape(128)
    valid = valid2.reshape(128)
    xs = _scatter_k(x2, p)
    ys = _run_ffn(valid, te, xs, W1, b1, W2, b2)
    out = _gather_k(ys, p)
    return out.reshape(1, S, D)


# speedup vs baseline: 1.4360x; 1.0559x over previous
import functools
import math

import jax
import jax.numpy as jnp
from jax import lax
from jax.experimental import pallas as pl
from jax.experimental.pallas import tpu as pltpu
from jax.experimental.pallas import tpu_sc as plsc

S = 2048
D = 1024
E = 8
FF = 2048
T = 128
SPAD = S + E * T
NT = SPAD // T
RB = 256
NRB = S // RB

NC, NS = 2, 16
NW = NC * NS
CHUNK = S // NW


def _router_body(x_ref, rw_ref, rb_ref, p_ref, comb_ref,
                 eid_s, rank_s, carry_s):
    b = pl.program_id(0)

    @pl.when(b == 0)
    def _():
        carry_s[...] = jnp.zeros_like(carry_s)

    logits = jnp.dot(x_ref[...], rw_ref[...],
                     preferred_element_type=jnp.float32) + rb_ref[...]
    m = jnp.max(logits, axis=1, keepdims=True)
    ex = jnp.exp(logits - m)
    probs = ex / jnp.sum(ex, axis=1, keepdims=True)
    eidc = jnp.argmax(probs, axis=1, keepdims=True).astype(jnp.int32)

    eiota = lax.broadcasted_iota(jnp.int32, (RB, E), 1)
    onehot = (eidc == eiota).astype(jnp.float32)

    eye = (lax.broadcasted_iota(jnp.int32, (RB, RB), 0) ==
           lax.broadcasted_iota(jnp.int32, (RB, RB), 1)).astype(jnp.float32)
    onehot_t = lax.dot_general(onehot, eye, (((0,), (0,)), ((), ())),
                               preferred_element_type=jnp.float32)

    srow = lax.broadcasted_iota(jnp.int32, (E, RB), 0).astype(jnp.float32)
    eid_row = jnp.sum(onehot_t * srow, axis=0, keepdims=True)

    ut = (lax.broadcasted_iota(jnp.int32, (RB, RB), 0) <
          lax.broadcasted_iota(jnp.int32, (RB, RB), 1)).astype(jnp.float32)
    inblk = lax.dot_general(onehot_t, ut, (((1,), (0,)), ((), ())),
                            preferred_element_type=jnp.float32)
    tot = inblk + carry_s[:, 0:1]
    rank_row = jnp.sum(onehot_t * tot, axis=0, keepdims=True)
    carry_s[:, 0:1] = carry_s[:, 0:1] + jnp.sum(onehot_t, axis=1,
                                                keepdims=True)

    eid2 = jnp.concatenate([eid_row[:, 0:128], eid_row[:, 128:256]], axis=0)
    rank2 = jnp.concatenate([rank_row[:, 0:128], rank_row[:, 128:256]],
                            axis=0)
    eid16 = jnp.concatenate([eid2] * 8, axis=0).astype(jnp.int32)
    rank16 = jnp.concatenate([rank2] * 8, axis=0)
    rhalf = lax.broadcasted_iota(jnp.int32, (16, 128), 0) // 2
    eid_s[...] = jnp.where(rhalf == b, eid16, eid_s[...])
    rank_s[...] = jnp.where(rhalf == b, rank16, rank_s[...])

    @pl.when(b == NRB - 1)
    def _():
        counts = carry_s[:, 0:1]
        padded = ((counts + (T - 1)) // T) * T
        lt = (lax.broadcasted_iota(jnp.int32, (E, E), 0) >
              lax.broadcasted_iota(jnp.int32, (E, E), 1)).astype(jnp.float32)
        offs = jnp.dot(lt, padded, preferred_element_type=jnp.float32)
        incl = offs + padded
        sel = lax.broadcasted_iota(jnp.int32, (E, 1), 0)

        eids = eid_s[...]
        p = rank_s[...]
        for e in range(E):
            off_e = jnp.sum(jnp.where(sel == e, offs, 0.0))
            p = p + jnp.where(eids == e, off_e, 0.0)
        p_ref[...] = p.astype(jnp.int32)

        tstart = lax.broadcasted_iota(jnp.int32, (1, 128), 1) * T
        te = jnp.zeros((1, 128), jnp.int32)
        for e in range(E):
            end_e = jnp.sum(jnp.where(sel == e, incl, 0.0))
            te = te + (tstart >= end_e.astype(jnp.int32)).astype(jnp.int32)
        total = jnp.sum(padded).astype(jnp.int32)
        valid = (tstart < total).astype(jnp.int32)
        comb_ref[...] = jnp.concatenate(
            [jnp.minimum(te, E - 1), valid,
             jnp.zeros((6, 128), jnp.int32)], axis=0)


def _run_router(x2, route_W, route_b):
    return pl.pallas_call(
        _router_body,
        grid=(NRB,),
        in_specs=[
            pl.BlockSpec((RB, D), lambda i: (i, 0)),
            pl.BlockSpec((D, E), lambda i: (0, 0)),
            pl.BlockSpec((1, E), lambda i: (0, 0)),
        ],
        out_specs=[
            pl.BlockSpec((16, 128), lambda i: (0, 0)),
            pl.BlockSpec((8, 128), lambda i: (0, 0)),
        ],
        out_shape=[
            jax.ShapeDtypeStruct((16, 128), jnp.int32),
            jax.ShapeDtypeStruct((8, 128), jnp.int32),
        ],
        scratch_shapes=[
            pltpu.VMEM((16, 128), jnp.int32),
            pltpu.VMEM((16, 128), jnp.float32),
            pltpu.VMEM((8, 128), jnp.float32),
        ],
    )(x2, route_W, route_b.reshape(1, E))


def _make_scatter():
    mesh = plsc.VectorSubcoreMesh(core_axis_name="c", subcore_axis_name="s")

    @functools.partial(
        pl.kernel,
        mesh=mesh,
        out_type=jax.ShapeDtypeStruct((SPAD, D), jnp.float32),
        scratch_types=[
            pltpu.VMEM((CHUNK,), jnp.int32),
            pltpu.VMEM((CHUNK, D), jnp.float32),
            pltpu.SemaphoreType.DMA,
        ],
    )
    def scatter_k(x_hbm, p_hbm, xs_hbm, idx_v, rows_v, sem):
        wid = lax.axis_index("s") * NC + lax.axis_index("c")
        base = wid * CHUNK
        pltpu.sync_copy(p_hbm.at[pl.ds(base, CHUNK)], idx_v)
        pltpu.sync_copy(x_hbm.at[pl.ds(base, CHUNK)], rows_v)
        pltpu.async_copy(rows_v, xs_hbm.at[idx_v], sem).wait()

    return scatter_k


def _ffn_body(c_ref, xs_ref, w1_ref, b1_ref, w2_ref, b2_ref, out_ref):
    t = pl.program_id(0)

    @pl.when(c_ref[128 + t] != 0)
    def _():
        te = c_ref[t]
        sel = lax.broadcasted_iota(jnp.int32, (E, 1), 0) == te
        b1r = jnp.sum(jnp.where(sel, b1_ref[...], 0.0), axis=0,
                      keepdims=True)
        b2r = jnp.sum(jnp.where(sel, b2_ref[...], 0.0), axis=0,
                      keepdims=True)
        h = jnp.dot(xs_ref[...], w1_ref[0],
                    preferred_element_type=jnp.float32) + b1r
        g = 0.5 * h * (1.0 + lax.erf(h * (1.0 / math.sqrt(2.0))))
        out_ref[...] = jnp.dot(g, w2_ref[0],
                               preferred_element_type=jnp.float32) + b2r


def _run_ffn(comb, xs, W1, b1, W2, b2):
    grid_spec = pltpu.PrefetchScalarGridSpec(
        num_scalar_prefetch=1,
        grid=(NT,),
        in_specs=[
            pl.BlockSpec((T, D), lambda t, c: (t, 0)),
            pl.BlockSpec((1, D, FF), lambda t, c: (c[t], 0, 0)),
            pl.BlockSpec((E, FF), lambda t, c: (0, 0)),
            pl.BlockSpec((1, FF, D), lambda t, c: (c[t], 0, 0)),
            pl.BlockSpec((E, D), lambda t, c: (0, 0)),
        ],
        out_specs=pl.BlockSpec((T, D), lambda t, c: (t, 0)),
    )
    return pl.pallas_call(
        _ffn_body,
        grid_spec=grid_spec,
        out_shape=jax.ShapeDtypeStruct((SPAD, D), jnp.float32),
        compiler_params=pltpu.CompilerParams(
            vmem_limit_bytes=100 * 1024 * 1024),
    )(comb, xs, W1, b1, W2, b2)


def _make_gather():
    mesh = plsc.VectorSubcoreMesh(core_axis_name="c", subcore_axis_name="s")

    @functools.partial(
        pl.kernel,
        mesh=mesh,
        out_type=jax.ShapeDtypeStruct((S, D), jnp.float32),
        scratch_types=[
            pltpu.VMEM((CHUNK,), jnp.int32),
            pltpu.VMEM((CHUNK, D), jnp.float32),
            pltpu.SemaphoreType.DMA,
        ],
    )
    def gather_k(ys_hbm, p_hbm, out_hbm, idx_v, rows_v, sem):
        wid = lax.axis_index("s") * NC + lax.axis_index("c")
        base = wid * CHUNK
        pltpu.sync_copy(p_hbm.at[pl.ds(base, CHUNK)], idx_v)
        pltpu.async_copy(ys_hbm.at[idx_v], rows_v, sem).wait()
        pltpu.sync_copy(rows_v, out_hbm.at[pl.ds(base, CHUNK)])

    return gather_k


_scatter_k = _make_scatter()
_gather_k = _make_gather()


def kernel(x, route_W, route_b, W1, b1, W2, b2):
    x2 = x.reshape(S, D)
    p2, comb2 = _run_router(x2, route_W, route_b)
    p = p2.reshape(S)
    comb = comb2.reshape(8 * 128)
    xs = _scatter_k(x2, p)
    ys = _run_ffn(comb, xs, W1, b1, W2, b2)
    out = _gather_k(ys, p)
    return out.reshape(1, S, D)
